# baseline (speedup 1.0000x reference)
import functools

import jax
import jax.numpy as jnp
from jax import lax
from jax.experimental import pallas as pl
from jax.experimental.pallas import tpu as pltpu
from jax.experimental.pallas import tpu_sc as plsc

D = 32
T = 200
NC = 2
NS = 16
NW = NC * NS
LANES = 16

TT = T // 8
BT = 4096 // 128
N_OUT = 4096 * T * D


@jax.jit
def _emb(idx_native, tok_padded, pos_table):
    mesh = plsc.VectorSubcoreMesh(core_axis_name="c", subcore_axis_name="s")

    @functools.partial(
        pl.kernel,
        out_type=jax.ShapeDtypeStruct((N_OUT,), jnp.float32),
        mesh=mesh,
        scratch_types=[
            pltpu.VMEM((8, 128), jnp.int32),
            pltpu.VMEM((1024, D), jnp.float32),
            pltpu.VMEM((4 * 4 * 8 * 128,), jnp.float32),
            pltpu.VMEM((T, D), jnp.float32),
            pltpu.SemaphoreType.DMA,
            pltpu.SemaphoreType.DMA,
        ],
        compiler_params=pltpu.CompilerParams(
            use_tc_tiling_on_sc=False, needs_layout_passes=False
        ),
    )
    def body(idx_hbm, tok_hbm, pos_hbm, out_hbm,
             idx_v, rows_v, tout_v, pos_v, gsem, ssem):
        w = lax.axis_index("s") * NC + lax.axis_index("c")
        pltpu.sync_copy(pos_hbm.at[pl.ds(0, T)], pos_v)

        lane = lax.iota(jnp.int32, LANES)
        off0 = (lane >> 3) * 1024 + (lane & 7) * 128
        off1 = ((lane + LANES) >> 3) * 1024 + (lane & 7) * 128

        def tt_body(tt, carry):
            pltpu.sync_copy(idx_hbm.at[tt, w], idx_v)
            copies = [
                pltpu.async_copy(
                    tok_hbm.at[idx_v.at[ti]],
                    rows_v.at[pl.ds(ti * 128, 128)],
                    gsem,
                )
                for ti in range(8)
            ]
            for c in copies:
                c.wait()

            for h in range(2):
                for k in range(4):
                    t = tt * 8 + h * 4 + k
                    rbase = (h * 4 + k) * 128
                    p0 = pos_v[t, pl.ds(0, LANES)]
                    p1 = pos_v[t, pl.ds(LANES, LANES)]
                    base0 = off0 + (k * 4096)
                    base1 = off1 + (k * 4096)

                    def bi_body(bi, c, rbase=rbase, p0=p0, p1=p1,
                                base0=base0, base1=base1):
                        bi_vec = jnp.full((LANES,), bi, jnp.int32)
                        v0 = rows_v[rbase + bi, pl.ds(0, LANES)] + p0
                        plsc.store_scatter(tout_v, [base0 + bi_vec], v0)
                        v1 = rows_v[rbase + bi, pl.ds(LANES, LANES)] + p1
                        plsc.store_scatter(tout_v, [base1 + bi_vec], v1)
                        return c

                    lax.fori_loop(0, 128, bi_body, 0, unroll=4)

                stores = []
                for k in range(4):
                    t = tt * 8 + h * 4 + k
                    for ct in range(4):
                        stores.append(pltpu.async_copy(
                            tout_v.at[pl.ds((k * 4 + ct) * 1024, 1024)],
                            out_hbm.at[pl.ds(((t * 4 + ct) * BT + w) * 1024,
                                             1024)],
                            ssem,
                        ))
                for s in stores:
                    s.wait()
            return carry

        lax.fori_loop(0, TT, tt_body, 0)

    return body(idx_native, tok_padded, pos_table)


def kernel(idx, tok_table, pos_table):
    idx = idx.astype(jnp.int32)
    idx_native = idx.T.reshape(TT, 8, BT, 128).transpose(0, 2, 1, 3)
    tok_lin = lax.optimization_barrier(tok_table.reshape(-1))
    q = _emb(idx_native, tok_lin.reshape(-1, D), pos_table)
    q = q.reshape(T, D // 8, BT, 8, 128)
    return q.transpose(2, 4, 0, 1, 3).reshape(4096, T, D)
